# Initial kernel scaffold; baseline (speedup 1.0000x reference)
import jax
import jax.numpy as jnp
from jax.experimental import pallas as pl


def kernel(x, mask):
    raise NotImplementedError("write your pallas kernel here")


# speedup vs baseline: 1.5757x; 1.5757x over previous
import functools

import jax
import jax.numpy as jnp
from jax import lax
from jax.experimental import pallas as pl
from jax.experimental.pallas import tpu as pltpu
from jax.experimental.pallas import tpu_sc as plsc

B, T, H = 16, 2048, 1024
NC, NS, L = 2, 16, 16
NW = NC * NS
C = 32
VPH = H // L

_MESH = plsc.VectorSubcoreMesh(
    core_axis_name="c", subcore_axis_name="s", num_cores=NC, num_subcores=NS
)


def _sc_body(x_hbm, mask_hbm, out_hbm, lens_hbm,
             mask_v, idx_v, rows_v, zeros_v, len_v, sem):
    wid = lax.axis_index("s") * NC + lax.axis_index("c")
    n = wid // 2
    half = wid % 2
    base = n * T

    zvec = jnp.zeros((L,), jnp.float32)

    def zbody(q, carry):
        r = q // VPH
        v = q % VPH
        zeros_v[r, pl.ds(v * L, L)] = zvec
        return carry
    lax.fori_loop(0, C * VPH, zbody, 0)

    pltpu.sync_copy(mask_hbm.at[n], mask_v)

    basevec = jnp.full((L,), base, jnp.int32)
    def fbody(i, carry):
        idx_v[pl.ds(i * L, L)] = basevec
        return carry
    lax.fori_loop(0, (T + L) // L, fbody, 0)

    def cbody(i, off):
        m = mask_v[pl.ds(i * L, L)]
        mb = m != 0
        cum = plsc.cumsum(m)
        tvec = lax.iota(jnp.int32, L) + (i * L + base)
        dest = off + cum - 1
        plsc.store_scatter(idx_v, [dest], tvec, mask=mb)
        return off + jnp.max(cum)
    length = lax.fori_loop(0, T // L, cbody, jnp.int32(0))

    @pl.when(half == 0)
    def _():
        len_v[...] = jnp.full((L,), length, jnp.int32)
        pltpu.sync_copy(len_v, lens_hbm.at[n])

    def gbody(i, carry):
        j0 = (2 * i + half) * C
        row0 = base + j0
        k = length - j0

        @pl.when(k > 0)
        def _():
            pltpu.async_copy(x_hbm.at[idx_v.at[pl.ds(j0, C)]], rows_v, sem).wait()

            @pl.when(k < C)
            def _():
                def zrow(r, c2):
                    for v in range(VPH):
                        rows_v[r, pl.ds(v * L, L)] = zvec
                    return c2
                lax.fori_loop(k, C, zrow, 0)

            pltpu.sync_copy(rows_v, out_hbm.at[pl.ds(row0, C)])

        @pl.when(k <= 0)
        def _():
            pltpu.sync_copy(zeros_v, out_hbm.at[pl.ds(row0, C)])

        return carry
    lax.fori_loop(0, T // (2 * C), gbody, 0)


_pad_masked = pl.kernel(
    _sc_body,
    out_type=(
        jax.ShapeDtypeStruct((B * T, H), jnp.float32),
        jax.ShapeDtypeStruct((B, L), jnp.int32),
    ),
    mesh=_MESH,
    compiler_params=pltpu.CompilerParams(needs_layout_passes=False),
    scratch_types=[
        pltpu.VMEM((T,), jnp.int32),
        pltpu.VMEM((T + L,), jnp.int32),
        pltpu.VMEM((C, H), jnp.float32),
        pltpu.VMEM((C, H), jnp.float32),
        pltpu.VMEM((L,), jnp.int32),
        pltpu.SemaphoreType.DMA,
    ],
)


@jax.jit
def kernel(x, mask):
    x2 = x.reshape(B * T, H)
    mask_i = mask.astype(jnp.int32)
    out2, lens2 = _pad_masked(x2, mask_i)
    return out2.reshape(B, T, H), lens2[:, 0]
